# baseline (speedup 1.0000x reference)
import functools

import jax
import jax.numpy as jnp
from jax import lax
from jax.experimental import pallas as pl
from jax.experimental.pallas import tpu as pltpu
from jax.experimental.pallas import tpu_sc as plsc

POS_OFFSET = 2
NC = 2
NS = 16
NW = NC * NS
LANES = 16
CH = 8
NSLOT = 3
GAHEAD = NSLOT - 1


@functools.partial(jax.jit, static_argnums=(4, 5))
def _embed_lookup(ids3, pos3, embed_tokens, embed_positions, n_tokens, d):
    rpw = n_tokens // NW
    nch = rpw // CH
    mesh = plsc.VectorSubcoreMesh(
        core_axis_name="c", subcore_axis_name="s",
        num_cores=NC, num_subcores=NS)

    @functools.partial(
        pl.kernel,
        out_type=jax.ShapeDtypeStruct((n_tokens, d), jnp.float32),
        mesh=mesh,
        scratch_types=[
            pltpu.VMEM((rpw,), jnp.int32),
            pltpu.VMEM((rpw,), jnp.int32),
            pltpu.VMEM((NSLOT, CH, d), jnp.float32),
            pltpu.VMEM((NSLOT, CH, d), jnp.float32),
            [pltpu.SemaphoreType.DMA] * NSLOT,
            [pltpu.SemaphoreType.DMA] * NSLOT,
            [pltpu.SemaphoreType.DMA] * NSLOT,
        ],
    )
    def body(ids_hbm, pos_hbm, tok_tab, pos_tab, out_hbm,
             idx_t, idx_p, buf_t, buf_p, sems_t, sems_p, sems_o):
        wid = lax.axis_index("s") * NC + lax.axis_index("c")
        pltpu.sync_copy(ids_hbm.at[wid], idx_t)
        pltpu.sync_copy(pos_hbm.at[wid], idx_p)
        for j in range(rpw // LANES):
            sl = pl.ds(j * LANES, LANES)
            idx_p[sl] = idx_p[sl] + POS_OFFSET
        base = wid * rpw

        def issue_gather(j):
            slot = j % NSLOT
            ct = pltpu.async_copy(
                tok_tab.at[idx_t.at[pl.ds(j * CH, CH)]],
                buf_t.at[slot], sems_t[slot])
            cp = pltpu.async_copy(
                pos_tab.at[idx_p.at[pl.ds(j * CH, CH)]],
                buf_p.at[slot], sems_p[slot])
            return ct, cp

        pending_g = {}
        pending_o = {}
        for j in range(min(GAHEAD, nch)):
            pending_g[j] = issue_gather(j)

        for j in range(nch):
            slot = j % NSLOT
            ct, cp = pending_g.pop(j)
            ct.wait()
            cp.wait()
            if j - 1 in pending_o:
                pending_o.pop(j - 1).wait()
            if j + GAHEAD < nch:
                pending_g[j + GAHEAD] = issue_gather(j + GAHEAD)

            @pl.loop(0, d // LANES)
            def _(i):
                off = pl.multiple_of(i * LANES, LANES)
                for r in range(CH):
                    buf_t[slot, r, pl.ds(off, LANES)] = (
                        buf_t[slot, r, pl.ds(off, LANES)]
                        + buf_p[slot, r, pl.ds(off, LANES)])

            pending_o[j] = pltpu.async_copy(
                buf_t.at[slot], out_hbm.at[pl.ds(base + j * CH, CH)],
                sems_o[slot])

        for j in sorted(pending_o):
            pending_o.pop(j).wait()

    return body(ids3, pos3, embed_tokens, embed_positions)


def kernel(input_ids, positions, embed_tokens, embed_positions):
    b, s = input_ids.shape
    d = embed_tokens.shape[1]
    n = b * s
    ids3 = input_ids.reshape(NW, n // NW).astype(jnp.int32)
    pos3 = positions.reshape(NW, n // NW).astype(jnp.int32)
    out = _embed_lookup(ids3, pos3, embed_tokens, embed_positions, n, d)
    return out.reshape(b, s, d)


# speedup vs baseline: 1.0188x; 1.0188x over previous
import functools

import jax
import jax.numpy as jnp
from jax import lax
from jax.experimental import pallas as pl
from jax.experimental.pallas import tpu as pltpu
from jax.experimental.pallas import tpu_sc as plsc

POS_OFFSET = 2
NC = 2
NS = 16
NW = NC * NS
LANES = 16
CH = 8
NSLOT = 3
GAHEAD = 2


@functools.partial(jax.jit, static_argnums=(4, 5))
def _embed_lookup(ids3, pos3, embed_tokens, embed_positions, n_tokens, d):
    rpw = n_tokens // NW
    nch = rpw // CH
    mesh = plsc.VectorSubcoreMesh(
        core_axis_name="c", subcore_axis_name="s",
        num_cores=NC, num_subcores=NS)

    @functools.partial(
        pl.kernel,
        out_type=jax.ShapeDtypeStruct((n_tokens, d), jnp.float32),
        mesh=mesh,
        scratch_types=[
            pltpu.VMEM((rpw,), jnp.int32),
            pltpu.VMEM((rpw,), jnp.int32),
            pltpu.VMEM((NSLOT, CH, d), jnp.float32),
            pltpu.VMEM((NSLOT, CH, d), jnp.float32),
            pltpu.VMEM((CH, d), jnp.float32),
            [pltpu.SemaphoreType.DMA] * NSLOT,
            [pltpu.SemaphoreType.DMA] * NSLOT,
            pltpu.SemaphoreType.DMA,
        ],
    )
    def body(ids_hbm, pos_hbm, tok_tab, pos_tab, out_hbm,
             idx_t, idx_p, buf_t, buf_p, buf_o, sems_t, sems_p, sem_o):
        wid = lax.axis_index("s") * NC + lax.axis_index("c")
        pltpu.sync_copy(ids_hbm.at[wid], idx_t)
        pltpu.sync_copy(pos_hbm.at[wid], idx_p)
        for j in range(rpw // LANES):
            sl = pl.ds(j * LANES, LANES)
            idx_p[sl] = idx_p[sl] + POS_OFFSET
        base = wid * rpw

        def issue_gather(j):
            slot = j % NSLOT
            ct = pltpu.async_copy(
                tok_tab.at[idx_t.at[pl.ds(j * CH, CH)]],
                buf_t.at[slot], sems_t[slot])
            cp = pltpu.async_copy(
                pos_tab.at[idx_p.at[pl.ds(j * CH, CH)]],
                buf_p.at[slot], sems_p[slot])
            return ct, cp

        pending_g = {}
        pending_o = {}
        for j in range(min(GAHEAD, nch)):
            pending_g[j] = issue_gather(j)

        for j in range(nch):
            slot = j % NSLOT
            ct, cp = pending_g.pop(j)
            ct.wait()
            cp.wait()
            if j + GAHEAD < nch:
                pending_g[j + GAHEAD] = issue_gather(j + GAHEAD)
            if j - 1 in pending_o:
                pending_o.pop(j - 1).wait()

            @pl.loop(0, d // LANES)
            def _(i):
                off = pl.multiple_of(i * LANES, LANES)
                for r in range(CH):
                    buf_o[r, pl.ds(off, LANES)] = (
                        buf_t[slot, r, pl.ds(off, LANES)]
                        + buf_p[slot, r, pl.ds(off, LANES)])

            pending_o[j] = pltpu.async_copy(
                buf_o, out_hbm.at[pl.ds(base + j * CH, CH)], sem_o)

        for j in sorted(pending_o):
            pending_o.pop(j).wait()

    return body(ids3, pos3, embed_tokens, embed_positions)


def kernel(input_ids, positions, embed_tokens, embed_positions):
    b, s = input_ids.shape
    d = embed_tokens.shape[1]
    n = b * s
    ids3 = input_ids.reshape(NW, n // NW).astype(jnp.int32)
    pos3 = positions.reshape(NW, n // NW).astype(jnp.int32)
    out = _embed_lookup(ids3, pos3, embed_tokens, embed_positions, n, d)
    return out.reshape(b, s, d)


# speedup vs baseline: 1.1822x; 1.1604x over previous
import functools

import jax
import jax.numpy as jnp
from jax import lax
from jax.experimental import pallas as pl
from jax.experimental.pallas import tpu as pltpu
from jax.experimental.pallas import tpu_sc as plsc

POS_OFFSET = 2
NC = 2
NS = 16
NW = NC * NS
LANES = 16
CH = 8
NSLOT_T = 3
NSLOT_P = 2
NSLOT_O = 2
GAHEAD = 2


@functools.partial(jax.jit, static_argnums=(4, 5))
def _embed_lookup(ids3, pos3, embed_tokens, embed_positions, n_tokens, d):
    rpw = n_tokens // NW
    nch = rpw // CH
    mesh = plsc.VectorSubcoreMesh(
        core_axis_name="c", subcore_axis_name="s",
        num_cores=NC, num_subcores=NS)

    @functools.partial(
        pl.kernel,
        out_type=jax.ShapeDtypeStruct((n_tokens, d), jnp.float32),
        mesh=mesh,
        scratch_types=[
            pltpu.VMEM((rpw,), jnp.int32),
            pltpu.VMEM((rpw,), jnp.int32),
            pltpu.VMEM((NSLOT_T, CH, d), jnp.float32),
            pltpu.VMEM((NSLOT_P, CH, d), jnp.float32),
            pltpu.VMEM((NSLOT_O, CH, d), jnp.float32),
            [pltpu.SemaphoreType.DMA] * NSLOT_T,
            [pltpu.SemaphoreType.DMA] * NSLOT_P,
            [pltpu.SemaphoreType.DMA] * NSLOT_O,
        ],
    )
    def body(ids_hbm, pos_hbm, tok_tab, pos_tab, out_hbm,
             idx_t, idx_p, buf_t, buf_p, buf_o, sems_t, sems_p, sems_o):
        wid = lax.axis_index("s") * NC + lax.axis_index("c")
        pltpu.sync_copy(ids_hbm.at[wid], idx_t)
        pltpu.sync_copy(pos_hbm.at[wid], idx_p)
        for j in range(rpw // LANES):
            sl = pl.ds(j * LANES, LANES)
            idx_p[sl] = idx_p[sl] + POS_OFFSET
        base = wid * rpw

        def issue_tok(j):
            return pltpu.async_copy(
                tok_tab.at[idx_t.at[pl.ds(j * CH, CH)]],
                buf_t.at[j % NSLOT_T], sems_t[j % NSLOT_T])

        def issue_pos(j):
            return pltpu.async_copy(
                pos_tab.at[idx_p.at[pl.ds(j * CH, CH)]],
                buf_p.at[j % NSLOT_P], sems_p[j % NSLOT_P])

        pending_t = {}
        pending_p = {}
        pending_o = {}
        for j in range(min(GAHEAD, nch)):
            pending_t[j] = issue_tok(j)
            pending_p[j] = issue_pos(j)

        for j in range(nch):
            ts, ps, os_ = j % NSLOT_T, j % NSLOT_P, j % NSLOT_O
            pending_t.pop(j).wait()
            pending_p.pop(j).wait()
            if j + GAHEAD < nch:
                pending_t[j + GAHEAD] = issue_tok(j + GAHEAD)
            if j - NSLOT_O in pending_o:
                pending_o.pop(j - NSLOT_O).wait()

            @pl.loop(0, d // LANES)
            def _(i):
                off = pl.multiple_of(i * LANES, LANES)
                for r in range(CH):
                    buf_o[os_, r, pl.ds(off, LANES)] = (
                        buf_t[ts, r, pl.ds(off, LANES)]
                        + buf_p[ps, r, pl.ds(off, LANES)])

            pending_o[j] = pltpu.async_copy(
                buf_o.at[os_], out_hbm.at[pl.ds(base + j * CH, CH)],
                sems_o[os_])
            if j + GAHEAD < nch:
                pending_p[j + GAHEAD] = issue_pos(j + GAHEAD)

        for j in sorted(pending_o):
            pending_o.pop(j).wait()

    return body(ids3, pos3, embed_tokens, embed_positions)


def kernel(input_ids, positions, embed_tokens, embed_positions):
    b, s = input_ids.shape
    d = embed_tokens.shape[1]
    n = b * s
    ids3 = input_ids.reshape(NW, n // NW).astype(jnp.int32)
    pos3 = positions.reshape(NW, n // NW).astype(jnp.int32)
    out = _embed_lookup(ids3, pos3, embed_tokens, embed_positions, n, d)
    return out.reshape(b, s, d)


# speedup vs baseline: 1.1862x; 1.0034x over previous
import functools

import jax
import jax.numpy as jnp
from jax import lax
from jax.experimental import pallas as pl
from jax.experimental.pallas import tpu as pltpu
from jax.experimental.pallas import tpu_sc as plsc

POS_OFFSET = 2
NC = 2
NS = 16
NW = NC * NS
LANES = 16
CH = 8
NSLOT_T = 3
NSLOT_P = 2
NSLOT_O = 2
GAHEAD = 2


@functools.partial(jax.jit, static_argnums=(4, 5))
def _embed_lookup(ids3, pos3, embed_tokens, embed_positions, n_tokens, d):
    rpw = n_tokens // NW
    nch = rpw // CH
    wps = ids3.shape[1] // rpw
    mesh = plsc.VectorSubcoreMesh(
        core_axis_name="c", subcore_axis_name="s",
        num_cores=NC, num_subcores=NS)

    @functools.partial(
        pl.kernel,
        out_type=jax.ShapeDtypeStruct((n_tokens, d), jnp.float32),
        mesh=mesh,
        scratch_types=[
            pltpu.VMEM((rpw,), jnp.int32),
            pltpu.VMEM((rpw,), jnp.int32),
            pltpu.VMEM((NSLOT_T, CH, d), jnp.float32),
            pltpu.VMEM((NSLOT_P, CH, d), jnp.float32),
            pltpu.VMEM((NSLOT_O, CH, d), jnp.float32),
            [pltpu.SemaphoreType.DMA] * NSLOT_T,
            [pltpu.SemaphoreType.DMA] * NSLOT_P,
            [pltpu.SemaphoreType.DMA] * NSLOT_O,
        ],
    )
    def body(ids_hbm, pos_hbm, tok_tab, pos_tab, out_hbm,
             idx_t, idx_p, buf_t, buf_p, buf_o, sems_t, sems_p, sems_o):
        wid = lax.axis_index("s") * NC + lax.axis_index("c")
        brow = wid // wps
        bcol = (wid % wps) * rpw
        pltpu.sync_copy(ids_hbm.at[brow, pl.ds(bcol, rpw)], idx_t)
        pltpu.sync_copy(pos_hbm.at[brow, pl.ds(bcol, rpw)], idx_p)
        for j in range(rpw // LANES):
            sl = pl.ds(j * LANES, LANES)
            idx_p[sl] = idx_p[sl] + POS_OFFSET
        base = wid * rpw

        def issue_tok(j):
            return pltpu.async_copy(
                tok_tab.at[idx_t.at[pl.ds(j * CH, CH)]],
                buf_t.at[j % NSLOT_T], sems_t[j % NSLOT_T])

        def issue_pos(j):
            return pltpu.async_copy(
                pos_tab.at[idx_p.at[pl.ds(j * CH, CH)]],
                buf_p.at[j % NSLOT_P], sems_p[j % NSLOT_P])

        pending_t = {}
        pending_p = {}
        pending_o = {}
        for j in range(min(GAHEAD, nch)):
            pending_t[j] = issue_tok(j)
            pending_p[j] = issue_pos(j)

        for j in range(nch):
            ts, ps, os_ = j % NSLOT_T, j % NSLOT_P, j % NSLOT_O
            pending_t.pop(j).wait()
            pending_p.pop(j).wait()
            if j + GAHEAD < nch:
                pending_t[j + GAHEAD] = issue_tok(j + GAHEAD)
            if j - NSLOT_O in pending_o:
                pending_o.pop(j - NSLOT_O).wait()

            @pl.loop(0, d // LANES)
            def _(i):
                off = pl.multiple_of(i * LANES, LANES)
                for r in range(CH):
                    buf_o[os_, r, pl.ds(off, LANES)] = (
                        buf_t[ts, r, pl.ds(off, LANES)]
                        + buf_p[ps, r, pl.ds(off, LANES)])

            pending_o[j] = pltpu.async_copy(
                buf_o.at[os_], out_hbm.at[pl.ds(base + j * CH, CH)],
                sems_o[os_])
            if j + GAHEAD < nch:
                pending_p[j + GAHEAD] = issue_pos(j + GAHEAD)

        for j in sorted(pending_o):
            pending_o.pop(j).wait()

    return body(ids3, pos3, embed_tokens, embed_positions)


def kernel(input_ids, positions, embed_tokens, embed_positions):
    b, s = input_ids.shape
    d = embed_tokens.shape[1]
    n = b * s
    out = _embed_lookup(input_ids, positions, embed_tokens, embed_positions,
                        n, d)
    return out.reshape(b, s, d)


# speedup vs baseline: 1.2594x; 1.0618x over previous
import functools

import jax
import jax.numpy as jnp
from jax import lax
from jax.experimental import pallas as pl
from jax.experimental.pallas import tpu as pltpu
from jax.experimental.pallas import tpu_sc as plsc

POS_OFFSET = 2
NC = 2
NS = 16
NW = NC * NS
LANES = 16
CH = 8


@functools.partial(jax.jit, static_argnums=(4, 5))
def _embed_lookup(ids2, pos2, embed_tokens, embed_positions, n_tokens, d):
    rpw = n_tokens // NW
    nch = rpw // CH
    wps = ids2.shape[1] // rpw
    mesh = plsc.VectorSubcoreMesh(
        core_axis_name="c", subcore_axis_name="s",
        num_cores=NC, num_subcores=NS)

    @functools.partial(
        pl.kernel,
        out_type=jax.ShapeDtypeStruct((n_tokens, d), jnp.float32),
        mesh=mesh,
        scratch_types=[
            pltpu.VMEM((rpw,), jnp.int32),
            pltpu.VMEM((rpw,), jnp.int32),
            pltpu.VMEM((2, CH, d), jnp.float32),
            pltpu.VMEM((2, CH, d), jnp.float32),
            pltpu.VMEM((2, CH, d), jnp.float32),
            [pltpu.SemaphoreType.DMA] * 2,
            [pltpu.SemaphoreType.DMA] * 2,
            [pltpu.SemaphoreType.DMA] * 2,
        ],
    )
    def body(ids_hbm, pos_hbm, tok_tab, pos_tab, out_hbm,
             idx_t, idx_p, buf_t, buf_p, buf_o, sems_t, sems_p, sems_o):
        wid = lax.axis_index("s") * NC + lax.axis_index("c")
        brow = wid // wps
        bcol = (wid % wps) * rpw
        pltpu.sync_copy(ids_hbm.at[brow, pl.ds(bcol, rpw)], idx_t)
        pltpu.sync_copy(pos_hbm.at[brow, pl.ds(bcol, rpw)], idx_p)

        @pl.loop(0, rpw // LANES)
        def _(j):
            sl = pl.ds(pl.multiple_of(j * LANES, LANES), LANES)
            idx_p[sl] = idx_p[sl] + POS_OFFSET

        base = wid * rpw

        def g_desc(c, slot):
            off = pl.multiple_of(c * CH, CH)
            ct = pltpu.make_async_copy(
                tok_tab.at[idx_t.at[pl.ds(off, CH)]],
                buf_t.at[slot], sems_t[slot])
            cp = pltpu.make_async_copy(
                pos_tab.at[idx_p.at[pl.ds(off, CH)]],
                buf_p.at[slot], sems_p[slot])
            return ct, cp

        def o_desc(c, slot):
            off = pl.multiple_of(c * CH, CH)
            return pltpu.make_async_copy(
                buf_o.at[slot], out_hbm.at[pl.ds(base + off, CH)],
                sems_o[slot])

        def issue_g(c, slot):
            ct, cp = g_desc(c, slot)
            ct.start()
            cp.start()

        def add_chunk(slot):
            @pl.loop(0, d // LANES)
            def _(i):
                off = pl.ds(pl.multiple_of(i * LANES, LANES), LANES)
                for r in range(CH):
                    buf_o[slot, r, off] = (
                        buf_t[slot, r, off] + buf_p[slot, r, off])

        issue_g(0, 0)
        issue_g(1, 1)

        def step(c, slot, first, last):
            ct, cp = g_desc(c, slot)
            ct.wait()
            cp.wait()

            @pl.when(jnp.logical_not(first))
            def _():
                o_desc(c - 2, slot).wait()

            add_chunk(slot)
            o_desc(c, slot).start()

            @pl.when(jnp.logical_not(last))
            def _():
                issue_g(c + 2, slot)

        @pl.loop(0, nch // 2)
        def _(i):
            a = pl.multiple_of(i * 2, 2)
            step(a, 0, i == 0, i == nch // 2 - 1)
            step(a + 1, 1, i == 0, i == nch // 2 - 1)

        o_desc(nch - 2, 0).wait()
        o_desc(nch - 1, 1).wait()

    return body(ids2, pos2, embed_tokens, embed_positions)


def kernel(input_ids, positions, embed_tokens, embed_positions):
    b, s = input_ids.shape
    d = embed_tokens.shape[1]
    n = b * s
    out = _embed_lookup(input_ids, positions, embed_tokens, embed_positions,
                        n, d)
    return out.reshape(b, s, d)
